# baseline (speedup 1.0000x reference)
import functools

import jax
import jax.numpy as jnp
from jax.experimental import pallas as pl
from jax.experimental.pallas import tpu as pltpu


def _linear_sigmoid_kernel(x_ref, w_ref, b_ref, o_ref):
    acc = jnp.dot(
        x_ref[...].astype(jnp.bfloat16),
        w_ref[...],
        preferred_element_type=jnp.float32,
    )
    o_ref[...] = jax.nn.sigmoid(acc + b_ref[...])


@jax.jit
def kernel(x, w_t, b2d):
    x = x.astype(jnp.float32)
    M, K = x.shape
    K2, N = w_t.shape
    assert K == K2 and b2d.shape == (1, N)

    w_bf = w_t.astype(jnp.bfloat16)
    b2d = b2d.astype(jnp.float32)

    tm = 2048
    while M % tm != 0 and tm > 8:
        tm //= 2
    m_pad = M
    if M % tm != 0:
        m_pad = ((M + tm - 1) // tm) * tm
        x = jnp.pad(x, ((0, m_pad - M), (0, 0)))

    out = pl.pallas_call(
        _linear_sigmoid_kernel,
        out_shape=jax.ShapeDtypeStruct((m_pad, N), jnp.float32),
        grid=(m_pad // tm,),
        in_specs=[
            pl.BlockSpec((tm, K), lambda i: (i, 0)),
            pl.BlockSpec((K, N), lambda i: (0, 0)),
            pl.BlockSpec((1, N), lambda i: (0, 0)),
        ],
        out_specs=pl.BlockSpec((tm, N), lambda i: (i, 0)),
        compiler_params=pltpu.CompilerParams(
            dimension_semantics=("parallel",),
        ),
    )(x, w_bf, b2d)

    if m_pad != M:
        out = out[:M]
    return out


# speedup vs baseline: 1.0894x; 1.0894x over previous
import functools

import jax
import jax.numpy as jnp
from jax.experimental import pallas as pl
from jax.experimental.pallas import tpu as pltpu


def _linear_sigmoid_kernel(x_ref, w_ref, b_ref, o_ref):
    acc = jnp.dot(
        x_ref[...].astype(jnp.bfloat16),
        w_ref[...].astype(jnp.bfloat16),
        preferred_element_type=jnp.float32,
    )
    o_ref[...] = jax.nn.sigmoid(acc + b_ref[...])


@jax.jit
def kernel(x, w_t, b2d):
    x = x.astype(jnp.float32)
    M, K = x.shape
    K2, N = w_t.shape
    assert K == K2 and b2d.shape == (1, N)

    w_t = w_t.astype(jnp.float32)
    b2d = b2d.astype(jnp.float32)

    tm = 2048
    while M % tm != 0 and tm > 8:
        tm //= 2
    m_pad = M
    if M % tm != 0:
        m_pad = ((M + tm - 1) // tm) * tm
        x = jnp.pad(x, ((0, m_pad - M), (0, 0)))

    out = pl.pallas_call(
        _linear_sigmoid_kernel,
        out_shape=jax.ShapeDtypeStruct((m_pad, N), jnp.float32),
        grid=(m_pad // tm,),
        in_specs=[
            pl.BlockSpec((tm, K), lambda i: (i, 0)),
            pl.BlockSpec((K, N), lambda i: (0, 0)),
            pl.BlockSpec((1, N), lambda i: (0, 0)),
        ],
        out_specs=pl.BlockSpec((tm, N), lambda i: (i, 0)),
        compiler_params=pltpu.CompilerParams(
            dimension_semantics=("parallel",),
        ),
    )(x, w_t, b2d)

    if m_pad != M:
        out = out[:M]
    return out


# speedup vs baseline: 1.1873x; 1.0899x over previous
import functools

import jax
import jax.numpy as jnp
from jax.experimental import pallas as pl
from jax.experimental.pallas import tpu as pltpu


def _linear_sigmoid_kernel(x_ref, w_ref, b_ref, o_ref, *, chunk):
    wb = w_ref[...].astype(jnp.bfloat16)
    b = b_ref[...]
    tm = x_ref.shape[0]
    for r in range(tm // chunk):
        xs = x_ref[pl.ds(r * chunk, chunk), :].astype(jnp.bfloat16)
        acc = jnp.dot(xs, wb, preferred_element_type=jnp.float32)
        o_ref[pl.ds(r * chunk, chunk), :] = jax.nn.sigmoid(acc + b)


@jax.jit
def kernel(x, w_t, b2d):
    x = x.astype(jnp.float32)
    M, K = x.shape
    K2, N = w_t.shape
    assert K == K2 and b2d.shape == (1, N)

    w_t = w_t.astype(jnp.float32)
    b2d = b2d.astype(jnp.float32)

    tm = 2048
    while M % tm != 0 and tm > 8:
        tm //= 2
    m_pad = M
    if M % tm != 0:
        m_pad = ((M + tm - 1) // tm) * tm
        x = jnp.pad(x, ((0, m_pad - M), (0, 0)))

    chunk = 256
    while tm % chunk != 0 and chunk > 8:
        chunk //= 2

    out = pl.pallas_call(
        functools.partial(_linear_sigmoid_kernel, chunk=chunk),
        out_shape=jax.ShapeDtypeStruct((m_pad, N), jnp.float32),
        grid=(m_pad // tm,),
        in_specs=[
            pl.BlockSpec((tm, K), lambda i: (i, 0)),
            pl.BlockSpec((K, N), lambda i: (0, 0)),
            pl.BlockSpec((1, N), lambda i: (0, 0)),
        ],
        out_specs=pl.BlockSpec((tm, N), lambda i: (i, 0)),
        compiler_params=pltpu.CompilerParams(
            dimension_semantics=("parallel",),
        ),
    )(x, w_t, b2d)

    if m_pad != M:
        out = out[:M]
    return out
